# baseline (speedup 1.0000x reference)
import functools

import jax
import jax.numpy as jnp
from jax import lax
from jax.experimental import pallas as pl
from jax.experimental.pallas import tpu as pltpu
from jax.experimental.pallas import tpu_sc as plsc

_NC = 2
_NS = 16
_CH = 128
_K = 8


def _seg_sum_partials(h, src3, dst3, zeros, np_):
    n, d = h.shape
    nch = src3.shape[1]
    rpt = np_ // _NS

    mesh = plsc.VectorSubcoreMesh(core_axis_name="c", subcore_axis_name="s")

    @functools.partial(
        pl.kernel,
        mesh=mesh,
        compiler_params=pltpu.CompilerParams(use_tc_tiling_on_sc=False),
        out_type=jax.ShapeDtypeStruct((_NC * np_, d), jnp.float32),
        scratch_types=[
            pltpu.VMEM((nch, _CH), jnp.int32),
            pltpu.VMEM((nch, _CH), jnp.int32),
            pltpu.VMEM((_K, _CH, d), jnp.float32),
            pltpu.VMEM_SHARED((np_, d), jnp.float32),
            pltpu.SemaphoreType.DMA((_K,)),
            pltpu.SemaphoreType.DMA,
        ],
    )
    def body(h_hbm, src_hbm, dst_hbm, z_hbm, out_hbm, sidx, didx, rows, acc,
             gsem, ssem):
        cid = lax.axis_index("c")
        sid = lax.axis_index("s")
        wid = sid * _NC + cid
        pltpu.sync_copy(z_hbm.at[pl.ds(sid * rpt, rpt)],
                        acc.at[pl.ds(sid * rpt, rpt)])
        pltpu.sync_copy(src_hbm.at[wid], sidx)
        pltpu.sync_copy(dst_hbm.at[wid], didx)
        plsc.subcore_barrier()

        def group(m, carry):
            base = m * _K
            gets = [pltpu.async_copy(h_hbm.at[sidx.at[base + b]], rows.at[b],
                                     gsem.at[b]) for b in range(_K)]
            puts = []
            for b in range(_K):
                gets[b].wait()
                puts.append(pltpu.async_copy(rows.at[b],
                                             acc.at[didx.at[base + b]],
                                             ssem, add=True))
            for p in puts:
                p.wait()
            return carry

        lax.fori_loop(0, nch // _K, group, 0)
        plsc.subcore_barrier()
        pltpu.sync_copy(acc.at[pl.ds(sid * rpt, rpt)],
                        out_hbm.at[pl.ds(cid * np_ + sid * rpt, rpt)])

    return body(h, src3, dst3, zeros)


_DOT = dict(preferred_element_type=jnp.float32, precision=lax.Precision.HIGHEST)


def _enc(x, w, b):
    n, d_in = x.shape
    d_h = w.shape[1]
    br = 1000

    def body(x_ref, w_ref, b_ref, o_ref):
        o_ref[...] = jnp.maximum(
            jnp.dot(x_ref[...], w_ref[...], **_DOT) + b_ref[...], 0.0)

    return pl.pallas_call(
        body,
        grid=(n // br,),
        in_specs=[
            pl.BlockSpec((br, d_in), lambda i: (i, 0)),
            pl.BlockSpec((d_in, d_h), lambda i: (0, 0)),
            pl.BlockSpec((1, d_h), lambda i: (0, 0)),
        ],
        out_specs=pl.BlockSpec((br, d_h), lambda i: (i, 0)),
        out_shape=jax.ShapeDtypeStruct((n, d_h), jnp.float32),
    )(x, w, b.reshape(1, d_h))


def _mlp_gate(p, h, s, w1, b1, w2, b2, gw, gb, dec_w=None, dec_b=None):
    n, d = h.shape
    br = 1000
    final = dec_w is not None
    d_out = dec_w.shape[1] if final else d

    def body(p_ref, h_ref, s_ref, w1_ref, b1_ref, w2_ref, b2_ref, gw_ref,
             gb_ref, *rest):
        z = p_ref[0] + p_ref[1] + h_ref[...]
        t = jnp.maximum(jnp.dot(z, w1_ref[...], **_DOT) + b1_ref[...], 0.0)
        conv = jnp.dot(t, w2_ref[...], **_DOT) + b2_ref[...]
        gl = (jnp.dot(s_ref[...], gw_ref[:d], **_DOT)
              + jnp.dot(conv, gw_ref[d:], **_DOT) + gb_ref[...])
        a = 1.0 / (1.0 + jnp.exp(-gl))
        ns = a * s_ref[...] + (1.0 - a) * conv
        if final:
            dw_ref, db_ref, o_ref = rest
            o_ref[...] = jnp.dot(ns, dw_ref[...], **_DOT) + db_ref[...]
        else:
            o1_ref, o2_ref = rest
            o1_ref[...] = ns
            o2_ref[...] = conv

    in_specs = [
        pl.BlockSpec((2, br, d), lambda i: (0, i, 0)),
        pl.BlockSpec((br, d), lambda i: (i, 0)),
        pl.BlockSpec((br, d), lambda i: (i, 0)),
        pl.BlockSpec((d, d), lambda i: (0, 0)),
        pl.BlockSpec((1, d), lambda i: (0, 0)),
        pl.BlockSpec((d, d), lambda i: (0, 0)),
        pl.BlockSpec((1, d), lambda i: (0, 0)),
        pl.BlockSpec((2 * d, 1), lambda i: (0, 0)),
        pl.BlockSpec((1, 1), lambda i: (0, 0)),
    ]
    args = [p, h, s, w1, b1.reshape(1, d), w2, b2.reshape(1, d), gw,
            gb.reshape(1, 1)]
    if final:
        in_specs += [
            pl.BlockSpec((d, d_out), lambda i: (0, 0)),
            pl.BlockSpec((1, d_out), lambda i: (0, 0)),
        ]
        args += [dec_w, dec_b.reshape(1, d_out)]
        out_specs = pl.BlockSpec((br, d_out), lambda i: (i, 0))
        out_shape = jax.ShapeDtypeStruct((n, d_out), jnp.float32)
    else:
        out_specs = [
            pl.BlockSpec((br, d), lambda i: (i, 0)),
            pl.BlockSpec((br, d), lambda i: (i, 0)),
        ]
        out_shape = [
            jax.ShapeDtypeStruct((n, d), jnp.float32),
            jax.ShapeDtypeStruct((n, d), jnp.float32),
        ]

    return pl.pallas_call(
        body,
        grid=(n // br,),
        in_specs=in_specs,
        out_specs=out_specs,
        out_shape=out_shape,
    )(*args)


def kernel(x, edge_index, enc_W, enc_b, gin0_W1, gin0_b1, gin0_W2, gin0_b2,
           gin1_W1, gin1_b1, gin1_W2, gin1_b2, gate_W, gate_b, dec_W, dec_b):
    n = x.shape[0]
    d = enc_W.shape[1]
    e = edge_index.shape[1]
    nw = _NC * _NS
    grp = nw * _CH * _K
    ep = ((e + grp - 1) // grp) * grp
    np_ = ((n + 16 * 8 - 1) // (16 * 8)) * (16 * 8)
    npad = ep - e
    src_p = jnp.concatenate(
        [edge_index[0], jnp.zeros((npad,), jnp.int32)])
    dst_p = jnp.concatenate(
        [edge_index[1], n + (jnp.arange(npad, dtype=jnp.int32) % (np_ - n))])
    nch = ep // (nw * _CH)
    src3 = src_p.reshape(nw, nch, _CH)
    dst3 = dst_p.reshape(nw, nch, _CH)
    zeros = jnp.zeros((np_, d), jnp.float32)

    init_x = _enc(x, enc_W, enc_b)

    p0 = _seg_sum_partials(init_x, src3, dst3, zeros, np_).reshape(2, np_, d)
    self_x, conv_x = _mlp_gate(p0, init_x, init_x, gin0_W1, gin0_b1,
                               gin0_W2, gin0_b2, gate_W, gate_b)
    p1 = _seg_sum_partials(conv_x, src3, dst3, zeros, np_).reshape(2, np_, d)
    return _mlp_gate(p1, conv_x, self_x, gin1_W1, gin1_b1, gin1_W2, gin1_b2,
                     gate_W, gate_b, dec_W, dec_b)


# speedup vs baseline: 2.9227x; 2.9227x over previous
import functools

import jax
import jax.numpy as jnp
from jax import lax
from jax.experimental import pallas as pl
from jax.experimental.pallas import tpu as pltpu
from jax.experimental.pallas import tpu_sc as plsc

_NC = 2
_NS = 16
_CH = 128
_K = 8


def _seg_sum_partials(h, src3, dst3, zeros, np_):
    n, d = h.shape
    nch = src3.shape[1]
    rpt = np_ // _NS

    mesh = plsc.VectorSubcoreMesh(core_axis_name="c", subcore_axis_name="s")

    @functools.partial(
        pl.kernel,
        mesh=mesh,
        compiler_params=pltpu.CompilerParams(use_tc_tiling_on_sc=False),
        out_type=jax.ShapeDtypeStruct((_NC * np_, d), jnp.float32),
        scratch_types=[
            pltpu.VMEM((nch, _CH), jnp.int32),
            pltpu.VMEM((nch, _CH), jnp.int32),
            pltpu.VMEM((_K, _CH, d), jnp.float32),
            pltpu.VMEM_SHARED((np_, d), jnp.float32),
            pltpu.SemaphoreType.DMA((_K,)),
            pltpu.SemaphoreType.DMA,
        ],
    )
    def body(h_hbm, src_hbm, dst_hbm, z_hbm, out_hbm, sidx, didx, rows, acc,
             gsem, ssem):
        cid = lax.axis_index("c")
        sid = lax.axis_index("s")
        wid = sid * _NC + cid
        pltpu.sync_copy(z_hbm.at[pl.ds(sid * rpt, rpt)],
                        acc.at[pl.ds(sid * rpt, rpt)])
        pltpu.sync_copy(src_hbm.at[wid], sidx)
        pltpu.sync_copy(dst_hbm.at[wid], didx)
        plsc.subcore_barrier()

        def group(m, carry):
            base = m * _K
            gets = [pltpu.async_copy(h_hbm.at[sidx.at[base + b]], rows.at[b],
                                     gsem.at[b]) for b in range(_K)]
            puts = []
            for b in range(_K):
                gets[b].wait()
                puts.append(pltpu.async_copy(rows.at[b],
                                             acc.at[didx.at[base + b]],
                                             ssem, add=True))
            for p in puts:
                p.wait()
            return carry

        lax.fori_loop(0, nch // _K, group, 0)
        plsc.subcore_barrier()
        pltpu.sync_copy(acc.at[pl.ds(sid * rpt, rpt)],
                        out_hbm.at[pl.ds(cid * np_ + sid * rpt, rpt)])

    return body(h, src3, dst3, zeros)


_DOT = dict(preferred_element_type=jnp.float32)


def _enc(x, w, b, np_):
    n, d_in = x.shape
    d_h = w.shape[1]
    br = np_ // 16

    def body(x_ref, w_ref, b_ref, o_ref):
        i = pl.program_id(0)
        row = i * br + jax.lax.broadcasted_iota(jnp.int32, (br, 1), 0)
        v = jnp.maximum(jnp.dot(x_ref[...], w_ref[...], **_DOT) + b_ref[...],
                        0.0)
        o_ref[...] = jnp.where(row < n, v, 0.0)

    return pl.pallas_call(
        body,
        grid=(np_ // br,),
        in_specs=[
            pl.BlockSpec((br, d_in), lambda i: (i, 0)),
            pl.BlockSpec((d_in, d_h), lambda i: (0, 0)),
            pl.BlockSpec((1, d_h), lambda i: (0, 0)),
        ],
        out_specs=pl.BlockSpec((br, d_h), lambda i: (i, 0)),
        out_shape=jax.ShapeDtypeStruct((np_, d_h), jnp.float32),
    )(x, w, b.reshape(1, d_h))


def _mlp_gate(p, h, s, w1, b1, w2, b2, gw, gb, dec_w=None, dec_b=None):
    np_, d = h.shape
    final = dec_w is not None
    n = 10000 if final else np_
    br = 1000 if final else np_ // 16
    d_out = dec_w.shape[1] if final else d

    def body(p_ref, h_ref, s_ref, w1_ref, b1_ref, w2_ref, b2_ref, gw_ref,
             gb_ref, *rest):
        z = p_ref[0] + p_ref[1] + h_ref[...]
        t = jnp.maximum(jnp.dot(z, w1_ref[...], **_DOT) + b1_ref[...], 0.0)
        conv = jnp.dot(t, w2_ref[...], **_DOT) + b2_ref[...]
        gl = (jnp.dot(s_ref[...], gw_ref[:d], **_DOT)
              + jnp.dot(conv, gw_ref[d:], **_DOT) + gb_ref[...])
        a = 1.0 / (1.0 + jnp.exp(-gl))
        ns = a * s_ref[...] + (1.0 - a) * conv
        if final:
            dw_ref, db_ref, o_ref = rest
            o_ref[...] = jnp.dot(ns, dw_ref[...], **_DOT) + db_ref[...]
        else:
            i = pl.program_id(0)
            row = i * br + jax.lax.broadcasted_iota(jnp.int32, (br, 1), 0)
            o1_ref, o2_ref = rest
            o1_ref[...] = ns
            o2_ref[...] = jnp.where(row < 10000, conv, 0.0)

    in_specs = [
        pl.BlockSpec((2, br, d), lambda i: (0, i, 0)),
        pl.BlockSpec((br, d), lambda i: (i, 0)),
        pl.BlockSpec((br, d), lambda i: (i, 0)),
        pl.BlockSpec((d, d), lambda i: (0, 0)),
        pl.BlockSpec((1, d), lambda i: (0, 0)),
        pl.BlockSpec((d, d), lambda i: (0, 0)),
        pl.BlockSpec((1, d), lambda i: (0, 0)),
        pl.BlockSpec((2 * d, 1), lambda i: (0, 0)),
        pl.BlockSpec((1, 1), lambda i: (0, 0)),
    ]
    args = [p, h, s, w1, b1.reshape(1, d), w2, b2.reshape(1, d), gw,
            gb.reshape(1, 1)]
    if final:
        in_specs += [
            pl.BlockSpec((d, d_out), lambda i: (0, 0)),
            pl.BlockSpec((1, d_out), lambda i: (0, 0)),
        ]
        args += [dec_w, dec_b.reshape(1, d_out)]
        out_specs = pl.BlockSpec((br, d_out), lambda i: (i, 0))
        out_shape = jax.ShapeDtypeStruct((n, d_out), jnp.float32)
    else:
        out_specs = [
            pl.BlockSpec((br, d), lambda i: (i, 0)),
            pl.BlockSpec((br, d), lambda i: (i, 0)),
        ]
        out_shape = [
            jax.ShapeDtypeStruct((np_, d), jnp.float32),
            jax.ShapeDtypeStruct((np_, d), jnp.float32),
        ]

    return pl.pallas_call(
        body,
        grid=(n // br,),
        in_specs=in_specs,
        out_specs=out_specs,
        out_shape=out_shape,
    )(*args)


def kernel(x, edge_index, enc_W, enc_b, gin0_W1, gin0_b1, gin0_W2, gin0_b2,
           gin1_W1, gin1_b1, gin1_W2, gin1_b2, gate_W, gate_b, dec_W, dec_b):
    n = x.shape[0]
    d = enc_W.shape[1]
    e = edge_index.shape[1]
    nw = _NC * _NS
    np_ = ((n + 16 * 8 - 1) // (16 * 8)) * (16 * 8)
    epw_r = e // nw
    grp = _CH * _K
    epw = ((epw_r + grp - 1) // grp) * grp
    ppw = epw - epw_r
    nch = epw // _CH
    src_pad = n + (jnp.arange(nw * ppw, dtype=jnp.int32) % (np_ - n))
    dst_pad = jnp.arange(nw * ppw, dtype=jnp.int32) % np_
    src3 = jnp.concatenate(
        [edge_index[0].reshape(nw, epw_r), src_pad.reshape(nw, ppw)],
        axis=1).reshape(nw, nch, _CH)
    dst3 = jnp.concatenate(
        [edge_index[1].reshape(nw, epw_r), dst_pad.reshape(nw, ppw)],
        axis=1).reshape(nw, nch, _CH)
    zeros = jnp.zeros((np_, d), jnp.float32)

    init_x = _enc(x, enc_W, enc_b, np_)

    p0 = _seg_sum_partials(init_x, src3, dst3, zeros, np_).reshape(2, np_, d)
    self_x, conv_x = _mlp_gate(p0, init_x, init_x, gin0_W1, gin0_b1,
                               gin0_W2, gin0_b2, gate_W, gate_b)
    p1 = _seg_sum_partials(conv_x, src3, dst3, zeros, np_).reshape(2, np_, d)
    return _mlp_gate(p1, conv_x, self_x, gin1_W1, gin1_b1, gin1_W2, gin1_b2,
                     gate_W, gate_b, dec_W, dec_b)


# speedup vs baseline: 3.3709x; 1.1533x over previous
import functools

import jax
import jax.numpy as jnp
from jax import lax
from jax.experimental import pallas as pl
from jax.experimental.pallas import tpu as pltpu
from jax.experimental.pallas import tpu_sc as plsc

_NC = 2
_NS = 16
_CH = 128
_K = 8


def _seg_sum_partials(h, src3, dst3, zeros, np_):
    n, d = h.shape
    nch = src3.shape[1]
    rpt = np_ // _NS

    mesh = plsc.VectorSubcoreMesh(core_axis_name="c", subcore_axis_name="s")

    @functools.partial(
        pl.kernel,
        mesh=mesh,
        compiler_params=pltpu.CompilerParams(use_tc_tiling_on_sc=False),
        out_type=jax.ShapeDtypeStruct((_NC * np_, d), jnp.float32),
        scratch_types=[
            pltpu.VMEM((nch, _CH), jnp.int32),
            pltpu.VMEM((nch, _CH), jnp.int32),
            pltpu.VMEM((_K, _CH, d), jnp.float32),
            pltpu.VMEM_SHARED((np_, d), jnp.float32),
            pltpu.SemaphoreType.DMA((_K,)),
            pltpu.SemaphoreType.DMA((_K,)),
        ],
    )
    def body(h_hbm, src_hbm, dst_hbm, z_hbm, out_hbm, sidx, didx, rows, acc,
             gsem, ssem):
        cid = lax.axis_index("c")
        sid = lax.axis_index("s")
        wid = sid * _NC + cid
        pltpu.sync_copy(z_hbm.at[pl.ds(sid * rpt, rpt)],
                        acc.at[pl.ds(sid * rpt, rpt)])
        pltpu.sync_copy(src_hbm.at[wid], sidx)
        pltpu.sync_copy(dst_hbm.at[wid], didx)

        def fire_gather(j, b):
            return pltpu.async_copy(h_hbm.at[sidx.at[j]], rows.at[b],
                                    gsem.at[b])

        def wait_gather(j, b):
            pltpu.make_async_copy(h_hbm.at[sidx.at[j]], rows.at[b],
                                  gsem.at[b]).wait()

        def fire_scatter(j, b):
            return pltpu.async_copy(rows.at[b], acc.at[didx.at[j]],
                                    ssem.at[b], add=True)

        def wait_scatter(j, b):
            pltpu.make_async_copy(rows.at[b], acc.at[didx.at[j]],
                                  ssem.at[b]).wait()

        for b in range(_K - 1):
            fire_gather(b, b)
        plsc.subcore_barrier()

        def group(m, carry):
            for b in range(_K):
                j = m * _K + b
                jj = j + _K - 1
                prev = (b - 1) % _K
                wait_gather(j, b)
                fire_scatter(j, b)

                @pl.when(jnp.logical_and(jj >= _K, jj < nch))
                def _():
                    wait_scatter(j - 1, prev)

                @pl.when(jnp.logical_and(jj >= _K - 1, jj < nch))
                def _():
                    fire_gather(jj, prev)
            return carry

        lax.fori_loop(0, nch // _K, group, 0)
        for i in range(_K):
            wait_scatter(nch - _K + i, i)
        plsc.subcore_barrier()
        pltpu.sync_copy(acc.at[pl.ds(sid * rpt, rpt)],
                        out_hbm.at[pl.ds(cid * np_ + sid * rpt, rpt)])

    return body(h, src3, dst3, zeros)


_DOT = dict(preferred_element_type=jnp.float32)


def _enc(x, w, b, np_):
    n, d_in = x.shape
    d_h = w.shape[1]
    br = np_ // 16

    def body(x_ref, w_ref, b_ref, o_ref):
        i = pl.program_id(0)
        row = i * br + jax.lax.broadcasted_iota(jnp.int32, (br, 1), 0)
        v = jnp.maximum(jnp.dot(x_ref[...], w_ref[...], **_DOT) + b_ref[...],
                        0.0)
        o_ref[...] = jnp.where(row < n, v, 0.0)

    return pl.pallas_call(
        body,
        grid=(np_ // br,),
        in_specs=[
            pl.BlockSpec((br, d_in), lambda i: (i, 0)),
            pl.BlockSpec((d_in, d_h), lambda i: (0, 0)),
            pl.BlockSpec((1, d_h), lambda i: (0, 0)),
        ],
        out_specs=pl.BlockSpec((br, d_h), lambda i: (i, 0)),
        out_shape=jax.ShapeDtypeStruct((np_, d_h), jnp.float32),
    )(x, w, b.reshape(1, d_h))


def _mlp_gate(p, h, s, w1, b1, w2, b2, gw, gb, dec_w=None, dec_b=None):
    np_, d = h.shape
    final = dec_w is not None
    n = 10000 if final else np_
    br = 1000 if final else np_ // 16
    d_out = dec_w.shape[1] if final else d

    def body(p_ref, h_ref, s_ref, w1_ref, b1_ref, w2_ref, b2_ref, gw_ref,
             gb_ref, *rest):
        z = p_ref[0] + p_ref[1] + h_ref[...]
        t = jnp.maximum(jnp.dot(z, w1_ref[...], **_DOT) + b1_ref[...], 0.0)
        conv = jnp.dot(t, w2_ref[...], **_DOT) + b2_ref[...]
        gl = (jnp.dot(s_ref[...], gw_ref[:d], **_DOT)
              + jnp.dot(conv, gw_ref[d:], **_DOT) + gb_ref[...])
        a = 1.0 / (1.0 + jnp.exp(-gl))
        ns = a * s_ref[...] + (1.0 - a) * conv
        if final:
            dw_ref, db_ref, o_ref = rest
            o_ref[...] = jnp.dot(ns, dw_ref[...], **_DOT) + db_ref[...]
        else:
            i = pl.program_id(0)
            row = i * br + jax.lax.broadcasted_iota(jnp.int32, (br, 1), 0)
            o1_ref, o2_ref = rest
            o1_ref[...] = ns
            o2_ref[...] = jnp.where(row < 10000, conv, 0.0)

    in_specs = [
        pl.BlockSpec((2, br, d), lambda i: (0, i, 0)),
        pl.BlockSpec((br, d), lambda i: (i, 0)),
        pl.BlockSpec((br, d), lambda i: (i, 0)),
        pl.BlockSpec((d, d), lambda i: (0, 0)),
        pl.BlockSpec((1, d), lambda i: (0, 0)),
        pl.BlockSpec((d, d), lambda i: (0, 0)),
        pl.BlockSpec((1, d), lambda i: (0, 0)),
        pl.BlockSpec((2 * d, 1), lambda i: (0, 0)),
        pl.BlockSpec((1, 1), lambda i: (0, 0)),
    ]
    args = [p, h, s, w1, b1.reshape(1, d), w2, b2.reshape(1, d), gw,
            gb.reshape(1, 1)]
    if final:
        in_specs += [
            pl.BlockSpec((d, d_out), lambda i: (0, 0)),
            pl.BlockSpec((1, d_out), lambda i: (0, 0)),
        ]
        args += [dec_w, dec_b.reshape(1, d_out)]
        out_specs = pl.BlockSpec((br, d_out), lambda i: (i, 0))
        out_shape = jax.ShapeDtypeStruct((n, d_out), jnp.float32)
    else:
        out_specs = [
            pl.BlockSpec((br, d), lambda i: (i, 0)),
            pl.BlockSpec((br, d), lambda i: (i, 0)),
        ]
        out_shape = [
            jax.ShapeDtypeStruct((np_, d), jnp.float32),
            jax.ShapeDtypeStruct((np_, d), jnp.float32),
        ]

    return pl.pallas_call(
        body,
        grid=(n // br,),
        in_specs=in_specs,
        out_specs=out_specs,
        out_shape=out_shape,
    )(*args)


def kernel(x, edge_index, enc_W, enc_b, gin0_W1, gin0_b1, gin0_W2, gin0_b2,
           gin1_W1, gin1_b1, gin1_W2, gin1_b2, gate_W, gate_b, dec_W, dec_b):
    n = x.shape[0]
    d = enc_W.shape[1]
    e = edge_index.shape[1]
    nw = _NC * _NS
    np_ = ((n + 16 * 8 - 1) // (16 * 8)) * (16 * 8)
    epw_r = e // nw
    grp = _CH * _K
    epw = ((epw_r + grp - 1) // grp) * grp
    ppw = epw - epw_r
    nch = epw // _CH
    src_pad = n + (jnp.arange(nw * ppw, dtype=jnp.int32) % (np_ - n))
    dst_pad = jnp.arange(nw * ppw, dtype=jnp.int32) % np_
    src3 = jnp.concatenate(
        [edge_index[0].reshape(nw, epw_r), src_pad.reshape(nw, ppw)],
        axis=1).reshape(nw, nch, _CH)
    dst3 = jnp.concatenate(
        [edge_index[1].reshape(nw, epw_r), dst_pad.reshape(nw, ppw)],
        axis=1).reshape(nw, nch, _CH)
    zeros = jnp.zeros((np_, d), jnp.float32)

    init_x = _enc(x, enc_W, enc_b, np_)

    p0 = _seg_sum_partials(init_x, src3, dst3, zeros, np_).reshape(2, np_, d)
    self_x, conv_x = _mlp_gate(p0, init_x, init_x, gin0_W1, gin0_b1,
                               gin0_W2, gin0_b2, gate_W, gate_b)
    p1 = _seg_sum_partials(conv_x, src3, dst3, zeros, np_).reshape(2, np_, d)
    return _mlp_gate(p1, conv_x, self_x, gin1_W1, gin1_b1, gin1_W2, gin1_b2,
                     gate_W, gate_b, dec_W, dec_b)


# speedup vs baseline: 3.6200x; 1.0739x over previous
import functools

import jax
import jax.numpy as jnp
from jax import lax
from jax.experimental import pallas as pl
from jax.experimental.pallas import tpu as pltpu
from jax.experimental.pallas import tpu_sc as plsc

_NC = 2
_NS = 16
_CH = 128
_K = 8


def _seg_sum_partials(h, src3, dst3, zeros, np_):
    n, d = h.shape
    nch = src3.shape[1]
    rpt = np_ // _NS

    mesh = plsc.VectorSubcoreMesh(core_axis_name="c", subcore_axis_name="s")

    @functools.partial(
        pl.kernel,
        mesh=mesh,
        compiler_params=pltpu.CompilerParams(use_tc_tiling_on_sc=False),
        out_type=jax.ShapeDtypeStruct((_NC * np_, d), jnp.float32),
        scratch_types=[
            pltpu.VMEM((nch, _CH), jnp.int32),
            pltpu.VMEM((nch, _CH), jnp.int32),
            pltpu.VMEM((_K, _CH, d), jnp.float32),
            pltpu.VMEM_SHARED((np_, d), jnp.float32),
            pltpu.SemaphoreType.DMA((_K,)),
            pltpu.SemaphoreType.DMA((_K,)),
        ],
    )
    def body(h_hbm, src_hbm, dst_hbm, z_hbm, out_hbm, sidx, didx, rows, acc,
             gsem, ssem):
        cid = lax.axis_index("c")
        sid = lax.axis_index("s")
        wid = sid * _NC + cid

        @pl.when(cid == 0)
        def _():
            pltpu.sync_copy(h_hbm.at[pl.ds(sid * rpt, rpt)],
                            acc.at[pl.ds(sid * rpt, rpt)])

        @pl.when(cid != 0)
        def _():
            pltpu.sync_copy(z_hbm.at[pl.ds(sid * rpt, rpt)],
                            acc.at[pl.ds(sid * rpt, rpt)])
        pltpu.sync_copy(src_hbm.at[wid], sidx)
        pltpu.sync_copy(dst_hbm.at[wid], didx)

        def fire_gather(j, b):
            return pltpu.async_copy(h_hbm.at[sidx.at[j]], rows.at[b],
                                    gsem.at[b])

        def wait_gather(j, b):
            pltpu.make_async_copy(h_hbm.at[sidx.at[j]], rows.at[b],
                                  gsem.at[b]).wait()

        def fire_scatter(j, b):
            return pltpu.async_copy(rows.at[b], acc.at[didx.at[j]],
                                    ssem.at[b], add=True)

        def wait_scatter(j, b):
            pltpu.make_async_copy(rows.at[b], acc.at[didx.at[j]],
                                  ssem.at[b]).wait()

        for b in range(_K - 1):
            fire_gather(b, b)
        plsc.subcore_barrier()

        def group(m, carry):
            for b in range(_K):
                j = m * _K + b
                jj = j + _K - 1
                prev = (b - 1) % _K
                wait_gather(j, b)
                fire_scatter(j, b)

                @pl.when(jnp.logical_and(jj >= _K, jj < nch))
                def _():
                    wait_scatter(j - 1, prev)

                @pl.when(jnp.logical_and(jj >= _K - 1, jj < nch))
                def _():
                    fire_gather(jj, prev)
            return carry

        lax.fori_loop(0, nch // _K, group, 0)
        for i in range(_K):
            wait_scatter(nch - _K + i, i)
        plsc.subcore_barrier()
        pltpu.sync_copy(acc.at[pl.ds(sid * rpt, rpt)],
                        out_hbm.at[pl.ds(cid * np_ + sid * rpt, rpt)])

    return body(h, src3, dst3, zeros)


_DOT = dict(preferred_element_type=jnp.float32)


def _enc(x, w, b, np_):
    n, d_in = x.shape
    d_h = w.shape[1]
    br = np_ // 8

    def body(x_ref, w_ref, b_ref, o_ref):
        i = pl.program_id(0)
        row = i * br + jax.lax.broadcasted_iota(jnp.int32, (br, 1), 0)
        v = jnp.maximum(jnp.dot(x_ref[...], w_ref[...], **_DOT) + b_ref[...],
                        0.0)
        o_ref[...] = jnp.where(row < n, v, 0.0)

    return pl.pallas_call(
        body,
        grid=(np_ // br,),
        in_specs=[
            pl.BlockSpec((br, d_in), lambda i: (i, 0)),
            pl.BlockSpec((d_in, d_h), lambda i: (0, 0)),
            pl.BlockSpec((1, d_h), lambda i: (0, 0)),
        ],
        out_specs=pl.BlockSpec((br, d_h), lambda i: (i, 0)),
        out_shape=jax.ShapeDtypeStruct((np_, d_h), jnp.float32),
    )(x, w, b.reshape(1, d_h))


def _mlp_gate(p, s, w1, b1, w2, b2, gw, gb, dec_w=None, dec_b=None):
    np_, d = s.shape
    final = dec_w is not None
    n = 10000 if final else np_
    br = 2000 if final else np_ // 8
    d_out = dec_w.shape[1] if final else d

    def body(p_ref, s_ref, w1_ref, b1_ref, w2_ref, b2_ref, gw_ref,
             gb_ref, *rest):
        z = p_ref[0] + p_ref[1]
        t = jnp.maximum(jnp.dot(z, w1_ref[...], **_DOT) + b1_ref[...], 0.0)
        conv = jnp.dot(t, w2_ref[...], **_DOT) + b2_ref[...]
        gl = (jnp.dot(s_ref[...], gw_ref[:d], **_DOT)
              + jnp.dot(conv, gw_ref[d:], **_DOT) + gb_ref[...])
        a = 1.0 / (1.0 + jnp.exp(-gl))
        ns = a * s_ref[...] + (1.0 - a) * conv
        if final:
            dw_ref, db_ref, o_ref = rest
            o_ref[...] = jnp.dot(ns, dw_ref[...], **_DOT) + db_ref[...]
        else:
            i = pl.program_id(0)
            row = i * br + jax.lax.broadcasted_iota(jnp.int32, (br, 1), 0)
            o1_ref, o2_ref = rest
            o1_ref[...] = ns
            o2_ref[...] = jnp.where(row < 10000, conv, 0.0)

    in_specs = [
        pl.BlockSpec((2, br, d), lambda i: (0, i, 0)),
        pl.BlockSpec((br, d), lambda i: (i, 0)),
        pl.BlockSpec((d, d), lambda i: (0, 0)),
        pl.BlockSpec((1, d), lambda i: (0, 0)),
        pl.BlockSpec((d, d), lambda i: (0, 0)),
        pl.BlockSpec((1, d), lambda i: (0, 0)),
        pl.BlockSpec((2 * d, 1), lambda i: (0, 0)),
        pl.BlockSpec((1, 1), lambda i: (0, 0)),
    ]
    args = [p, s, w1, b1.reshape(1, d), w2, b2.reshape(1, d), gw,
            gb.reshape(1, 1)]
    if final:
        in_specs += [
            pl.BlockSpec((d, d_out), lambda i: (0, 0)),
            pl.BlockSpec((1, d_out), lambda i: (0, 0)),
        ]
        args += [dec_w, dec_b.reshape(1, d_out)]
        out_specs = pl.BlockSpec((br, d_out), lambda i: (i, 0))
        out_shape = jax.ShapeDtypeStruct((n, d_out), jnp.float32)
    else:
        out_specs = [
            pl.BlockSpec((br, d), lambda i: (i, 0)),
            pl.BlockSpec((br, d), lambda i: (i, 0)),
        ]
        out_shape = [
            jax.ShapeDtypeStruct((np_, d), jnp.float32),
            jax.ShapeDtypeStruct((np_, d), jnp.float32),
        ]

    return pl.pallas_call(
        body,
        grid=(n // br,),
        in_specs=in_specs,
        out_specs=out_specs,
        out_shape=out_shape,
    )(*args)


def kernel(x, edge_index, enc_W, enc_b, gin0_W1, gin0_b1, gin0_W2, gin0_b2,
           gin1_W1, gin1_b1, gin1_W2, gin1_b2, gate_W, gate_b, dec_W, dec_b):
    n = x.shape[0]
    d = enc_W.shape[1]
    e = edge_index.shape[1]
    nw = _NC * _NS
    np_ = ((n + 16 * 8 - 1) // (16 * 8)) * (16 * 8)
    epw_r = e // nw
    grp = _CH * _K
    epw = ((epw_r + grp - 1) // grp) * grp
    ppw = epw - epw_r
    nch = epw // _CH
    src_pad = n + (jnp.arange(nw * ppw, dtype=jnp.int32) % (np_ - n))
    dst_pad = jnp.arange(nw * ppw, dtype=jnp.int32) % np_
    src3 = jnp.concatenate(
        [edge_index[0].reshape(nw, epw_r), src_pad.reshape(nw, ppw)],
        axis=1).reshape(nw, nch, _CH)
    dst3 = jnp.concatenate(
        [edge_index[1].reshape(nw, epw_r), dst_pad.reshape(nw, ppw)],
        axis=1).reshape(nw, nch, _CH)
    zeros = jnp.zeros((np_, d), jnp.float32)

    init_x = _enc(x, enc_W, enc_b, np_)

    p0 = _seg_sum_partials(init_x, src3, dst3, zeros, np_).reshape(2, np_, d)
    self_x, conv_x = _mlp_gate(p0, init_x, gin0_W1, gin0_b1,
                               gin0_W2, gin0_b2, gate_W, gate_b)
    p1 = _seg_sum_partials(conv_x, src3, dst3, zeros, np_).reshape(2, np_, d)
    return _mlp_gate(p1, self_x, gin1_W1, gin1_b1, gin1_W2, gin1_b2,
                     gate_W, gate_b, dec_W, dec_b)


# speedup vs baseline: 4.2175x; 1.1650x over previous
import functools

import jax
import jax.numpy as jnp
from jax import lax
from jax.experimental import pallas as pl
from jax.experimental.pallas import tpu as pltpu
from jax.experimental.pallas import tpu_sc as plsc

_NC = 2
_NS = 16
_CH = 128
_K = 8


def _seg_sum_partials(h, src3, dst3, zeros, np_):
    n, d = h.shape
    nch = src3.shape[1]
    rpt = np_ // _NS

    mesh = plsc.VectorSubcoreMesh(core_axis_name="c", subcore_axis_name="s")

    @functools.partial(
        pl.kernel,
        mesh=mesh,
        compiler_params=pltpu.CompilerParams(use_tc_tiling_on_sc=False),
        out_type=jax.ShapeDtypeStruct((_NC * np_, d), jnp.float32),
        scratch_types=[
            pltpu.VMEM((nch, _CH), jnp.int32),
            pltpu.VMEM((nch, _CH), jnp.int32),
            pltpu.VMEM((_K, _CH, d), jnp.float32),
            pltpu.VMEM_SHARED((np_, d), jnp.float32),
            pltpu.SemaphoreType.DMA((_K,)),
            pltpu.SemaphoreType.DMA((_K,)),
        ],
    )
    def body(h_hbm, src_hbm, dst_hbm, z_hbm, out_hbm, sidx, didx, rows, acc,
             gsem, ssem):
        cid = lax.axis_index("c")
        sid = lax.axis_index("s")
        wid = sid * _NC + cid

        @pl.when(cid == 0)
        def _():
            pltpu.sync_copy(h_hbm.at[pl.ds(sid * rpt, rpt)],
                            acc.at[pl.ds(sid * rpt, rpt)])

        @pl.when(cid != 0)
        def _():
            pltpu.sync_copy(z_hbm.at[pl.ds(sid * rpt, rpt)],
                            acc.at[pl.ds(sid * rpt, rpt)])
        pltpu.sync_copy(src_hbm.at[wid], sidx)
        pltpu.sync_copy(dst_hbm.at[wid], didx)

        def fire_gather(j, b):
            return pltpu.async_copy(h_hbm.at[sidx.at[j]], rows.at[b],
                                    gsem.at[b])

        def wait_gather(j, b):
            pltpu.make_async_copy(h_hbm.at[sidx.at[j]], rows.at[b],
                                  gsem.at[b]).wait()

        def fire_scatter(j, b):
            return pltpu.async_copy(rows.at[b], acc.at[didx.at[j]],
                                    ssem.at[b], add=True)

        def wait_scatter(j, b):
            pltpu.make_async_copy(rows.at[b], acc.at[didx.at[j]],
                                  ssem.at[b]).wait()

        for b in range(_K - 1):
            fire_gather(b, b)
        plsc.subcore_barrier()

        def group(m, carry):
            for b in range(_K):
                j = m * _K + b
                jj = j + _K - 1
                prev = (b - 1) % _K
                wait_gather(j, b)
                fire_scatter(j, b)

                @pl.when(jnp.logical_and(jj >= _K, jj < nch))
                def _():
                    wait_scatter(j - 1, prev)

                @pl.when(jnp.logical_and(jj >= _K - 1, jj < nch))
                def _():
                    fire_gather(jj, prev)
            return carry

        lax.fori_loop(0, nch // _K, group, 0)
        for i in range(_K):
            wait_scatter(nch - _K + i, i)
        plsc.subcore_barrier()
        pltpu.sync_copy(acc.at[pl.ds(sid * rpt, rpt)],
                        out_hbm.at[pl.ds(cid * np_ + sid * rpt, rpt)])

    return body(h, src3, dst3, zeros)


_DOT = dict(preferred_element_type=jnp.float32)


def _enc(x, w, b, np_):
    n, d_in = x.shape
    d_h = w.shape[1]
    npr = np_ // 2
    br = npr // 8
    nb = 8

    def body(xt_ref, xb_ref, w_ref, b_ref, o_ref):
        i = pl.program_id(0)
        row = i * br + jax.lax.broadcasted_iota(jnp.int32, (br, 1), 0)
        ot = jnp.maximum(
            jnp.dot(xt_ref[...], w_ref[...], **_DOT) + b_ref[...], 0.0)
        ob = jnp.maximum(
            jnp.dot(xb_ref[...], w_ref[...], **_DOT) + b_ref[...], 0.0)
        ob = jnp.where(row < n - npr, ob, 0.0)
        o_ref[...] = jax.lax.concatenate([ot, ob], 1)

    return pl.pallas_call(
        body,
        grid=(nb,),
        in_specs=[
            pl.BlockSpec((br, d_in), lambda i: (i, 0)),
            pl.BlockSpec((br, d_in), lambda i: (i + nb, 0)),
            pl.BlockSpec((d_in, d_h), lambda i: (0, 0)),
            pl.BlockSpec((1, d_h), lambda i: (0, 0)),
        ],
        out_specs=pl.BlockSpec((br, 2 * d_h), lambda i: (i, 0)),
        out_shape=jax.ShapeDtypeStruct((npr, 2 * d_h), jnp.float32),
    )(x, x, w, b.reshape(1, d_h))


def _bd(w):
    z = jnp.zeros_like(w)
    return jnp.concatenate([jnp.concatenate([w, z], axis=1),
                            jnp.concatenate([z, w], axis=1)], axis=0)


def _gate_mat(gv, d):
    g = jnp.broadcast_to(gv, (d, d))
    z = jnp.zeros((d, d), jnp.float32)
    return jnp.concatenate([jnp.concatenate([g, z], axis=1),
                            jnp.concatenate([z, g], axis=1)], axis=0)


def _mlp_gate(p, s, w1, b1, w2, b2, gw, gb, n, dec_w=None, dec_b=None):
    npr, d2 = s.shape
    d = d2 // 2
    final = dec_w is not None
    br = npr // 8

    def body(p_ref, s_ref, w1_ref, b1_ref, w2_ref, b2_ref, gws_ref,
             gwc_ref, gb_ref, *rest):
        z = p_ref[0] + p_ref[1]
        t = jnp.maximum(jnp.dot(z, w1_ref[...], **_DOT) + b1_ref[...], 0.0)
        conv = jnp.dot(t, w2_ref[...], **_DOT) + b2_ref[...]
        gl = (jnp.dot(s_ref[...], gws_ref[...], **_DOT)
              + jnp.dot(conv, gwc_ref[...], **_DOT) + gb_ref[...])
        a = 1.0 / (1.0 + jnp.exp(-gl))
        ns = a * s_ref[...] + (1.0 - a) * conv
        if final:
            dw_ref, db_ref, o_ref = rest
            o_ref[...] = jnp.dot(ns, dw_ref[...], **_DOT) + db_ref[...]
        else:
            i = pl.program_id(0)
            row = i * br + jax.lax.broadcasted_iota(jnp.int32, (br, 1), 0)
            o1_ref, o2_ref = rest
            o1_ref[...] = ns
            cb = jnp.where(row < n - npr, conv[:, d:], 0.0)
            o2_ref[...] = jax.lax.concatenate([conv[:, :d], cb], 1)

    full = lambda shp: pl.BlockSpec(shp, lambda i: tuple(0 for _ in shp))
    in_specs = [
        pl.BlockSpec((2, br, d2), lambda i: (0, i, 0)),
        pl.BlockSpec((br, d2), lambda i: (i, 0)),
        full((d2, d2)),
        full((1, d2)),
        full((d2, d2)),
        full((1, d2)),
        full((d2, d2)),
        full((d2, d2)),
        full((1, 1)),
    ]
    b2x = lambda b: jnp.concatenate([b, b]).reshape(1, d2)
    args = [p, s, _bd(w1), b2x(b1), _bd(w2), b2x(b2),
            _gate_mat(gw[:d], d), _gate_mat(gw[d:], d), gb.reshape(1, 1)]
    if final:
        d_out = dec_w.shape[1]
        in_specs += [full((d2, 2 * d_out)), full((1, 2 * d_out))]
        args += [_bd(dec_w), b2x(dec_b)]
        out_specs = pl.BlockSpec((br, 2 * d_out), lambda i: (i, 0))
        out_shape = jax.ShapeDtypeStruct((npr, 2 * d_out), jnp.float32)
    else:
        out_specs = [
            pl.BlockSpec((br, d2), lambda i: (i, 0)),
            pl.BlockSpec((br, d2), lambda i: (i, 0)),
        ]
        out_shape = [
            jax.ShapeDtypeStruct((npr, d2), jnp.float32),
            jax.ShapeDtypeStruct((npr, d2), jnp.float32),
        ]

    return pl.pallas_call(
        body,
        grid=(npr // br,),
        in_specs=in_specs,
        out_specs=out_specs,
        out_shape=out_shape,
    )(*args)


def kernel(x, edge_index, enc_W, enc_b, gin0_W1, gin0_b1, gin0_W2, gin0_b2,
           gin1_W1, gin1_b1, gin1_W2, gin1_b2, gate_W, gate_b, dec_W, dec_b):
    n = x.shape[0]
    d = enc_W.shape[1]
    e = edge_index.shape[1]
    nw = _NC * _NS
    np_ = ((n + 16 * 8 - 1) // (16 * 8)) * (16 * 8)
    npr = np_ // 2
    epw_r = e // nw
    grp = _CH * _K
    epw = ((epw_r + grp - 1) // grp) * grp
    ppw = epw - epw_r
    nch = epw // _CH
    src_pad = n + (jnp.arange(nw * ppw, dtype=jnp.int32) % (np_ - n))
    dst_pad = jnp.arange(nw * ppw, dtype=jnp.int32) % np_

    def remap(r):
        return jnp.where(r < npr, 2 * r, 2 * r - np_ + 1)

    src3 = remap(jnp.concatenate(
        [edge_index[0].reshape(nw, epw_r), src_pad.reshape(nw, ppw)],
        axis=1)).reshape(nw, nch, _CH)
    dst3 = remap(jnp.concatenate(
        [edge_index[1].reshape(nw, epw_r), dst_pad.reshape(nw, ppw)],
        axis=1)).reshape(nw, nch, _CH)
    zeros = jnp.zeros((np_, d), jnp.float32)

    init_pair = _enc(x, enc_W, enc_b, np_)

    p0 = _seg_sum_partials(init_pair.reshape(np_, d), src3, dst3, zeros,
                           np_).reshape(2, npr, 2 * d)
    self_pair, conv_pair = _mlp_gate(p0, init_pair, gin0_W1, gin0_b1,
                                     gin0_W2, gin0_b2, gate_W, gate_b, n)
    p1 = _seg_sum_partials(conv_pair.reshape(np_, d), src3, dst3, zeros,
                           np_).reshape(2, npr, 2 * d)
    out_pair = _mlp_gate(p1, self_pair, gin1_W1, gin1_b1, gin1_W2, gin1_b2,
                         gate_W, gate_b, n, dec_W, dec_b)
    d_out = dec_W.shape[1]
    return jnp.concatenate([out_pair[:, :d_out], out_pair[:, d_out:]],
                           axis=0)[:n]


# speedup vs baseline: 4.5023x; 1.0675x over previous
import functools

import jax
import jax.numpy as jnp
from jax import lax
from jax.experimental import pallas as pl
from jax.experimental.pallas import tpu as pltpu
from jax.experimental.pallas import tpu_sc as plsc

_NC = 2
_NS = 16
_CH = 128
_K = 8


def _seg_sum_partials(h, ei4, zeros, np_):
    n, d = h.shape
    nch = ei4.shape[2]
    rpt = np_ // _NS

    mesh = plsc.VectorSubcoreMesh(core_axis_name="c", subcore_axis_name="s")

    @functools.partial(
        pl.kernel,
        mesh=mesh,
        compiler_params=pltpu.CompilerParams(use_tc_tiling_on_sc=False),
        out_type=jax.ShapeDtypeStruct((_NC * np_, d), jnp.float32),
        scratch_types=[
            pltpu.VMEM((nch, _CH), jnp.int32),
            pltpu.VMEM((nch, _CH), jnp.int32),
            pltpu.VMEM((_K, _CH, d), jnp.float32),
            pltpu.VMEM_SHARED((np_, d), jnp.float32),
            pltpu.SemaphoreType.DMA((_K,)),
            pltpu.SemaphoreType.DMA((_K,)),
        ],
    )
    def body(h_hbm, ei_hbm, z_hbm, out_hbm, sidx, didx, rows, acc,
             gsem, ssem):
        cid = lax.axis_index("c")
        sid = lax.axis_index("s")
        wid = sid * _NC + cid

        @pl.when(cid == 0)
        def _():
            pltpu.sync_copy(h_hbm.at[pl.ds(sid * rpt, rpt)],
                            acc.at[pl.ds(sid * rpt, rpt)])

        @pl.when(cid != 0)
        def _():
            pltpu.sync_copy(z_hbm.at[pl.ds(sid * rpt, rpt)],
                            acc.at[pl.ds(sid * rpt, rpt)])
        pltpu.sync_copy(ei_hbm.at[0, wid], sidx)
        pltpu.sync_copy(ei_hbm.at[1, wid], didx)

        def fire_gather(j, b):
            return pltpu.async_copy(h_hbm.at[sidx.at[j]], rows.at[b],
                                    gsem.at[b])

        def wait_gather(j, b):
            pltpu.make_async_copy(h_hbm.at[sidx.at[j]], rows.at[b],
                                  gsem.at[b]).wait()

        def fire_scatter(j, b):
            return pltpu.async_copy(rows.at[b], acc.at[didx.at[j]],
                                    ssem.at[b], add=True)

        def wait_scatter(j, b):
            pltpu.make_async_copy(rows.at[b], acc.at[didx.at[j]],
                                  ssem.at[b]).wait()

        for b in range(_K - 1):
            fire_gather(b, b)
        plsc.subcore_barrier()

        def group(m, carry):
            for b in range(_K):
                j = m * _K + b
                jj = j + _K - 1
                prev = (b - 1) % _K
                wait_gather(j, b)
                fire_scatter(j, b)

                @pl.when(jnp.logical_and(jj >= _K, jj < nch))
                def _():
                    wait_scatter(j - 1, prev)

                @pl.when(jnp.logical_and(jj >= _K - 1, jj < nch))
                def _():
                    fire_gather(jj, prev)
            return carry

        lax.fori_loop(0, nch // _K, group, 0)
        for i in range(_K):
            wait_scatter(nch - _K + i, i)
        plsc.subcore_barrier()
        pltpu.sync_copy(acc.at[pl.ds(sid * rpt, rpt)],
                        out_hbm.at[pl.ds(cid * np_ + sid * rpt, rpt)])

    return body(h, ei4, zeros)


_DOT = dict(preferred_element_type=jnp.float32)


def _enc(x, w, b, np_):
    n, d_in = x.shape
    d_h = w.shape[1]
    npr = np_ // 2
    br = npr // 8
    nb = 8

    def body(xt_ref, xb_ref, w_ref, b_ref, o_ref):
        i = pl.program_id(0)
        row = i * br + jax.lax.broadcasted_iota(jnp.int32, (br, 1), 0)
        ot = jnp.maximum(
            jnp.dot(xt_ref[...], w_ref[...], **_DOT) + b_ref[...], 0.0)
        ob = jnp.maximum(
            jnp.dot(xb_ref[...], w_ref[...], **_DOT) + b_ref[...], 0.0)
        ob = jnp.where(row < n - npr, ob, 0.0)
        o_ref[...] = jax.lax.concatenate([ot, ob], 1)

    return pl.pallas_call(
        body,
        grid=(nb,),
        in_specs=[
            pl.BlockSpec((br, d_in), lambda i: (i, 0)),
            pl.BlockSpec((br, d_in), lambda i: (i + nb, 0)),
            pl.BlockSpec((d_in, d_h), lambda i: (0, 0)),
            pl.BlockSpec((1, d_h), lambda i: (0, 0)),
        ],
        out_specs=pl.BlockSpec((br, 2 * d_h), lambda i: (i, 0)),
        out_shape=jax.ShapeDtypeStruct((npr, 2 * d_h), jnp.float32),
    )(x, x, w, b.reshape(1, d_h))


def _bd(w):
    z = jnp.zeros_like(w)
    return jnp.concatenate([jnp.concatenate([w, z], axis=1),
                            jnp.concatenate([z, w], axis=1)], axis=0)


def _gate_mat(gv, d):
    g = jnp.broadcast_to(gv, (d, d))
    z = jnp.zeros((d, d), jnp.float32)
    return jnp.concatenate([jnp.concatenate([g, z], axis=1),
                            jnp.concatenate([z, g], axis=1)], axis=0)


def _mlp_gate(p, s, w1, b1, w2, b2, gw, gb, n, dec_w=None, dec_b=None):
    npr, d2 = s.shape
    d = d2 // 2
    final = dec_w is not None
    br = npr // 8

    def body(p_ref, s_ref, w1_ref, b1_ref, w2_ref, b2_ref, gws_ref,
             gwc_ref, gb_ref, *rest):
        z = p_ref[0] + p_ref[1]
        t = jnp.maximum(jnp.dot(z, w1_ref[...], **_DOT) + b1_ref[...], 0.0)
        conv = jnp.dot(t, w2_ref[...], **_DOT) + b2_ref[...]
        gl = (jnp.dot(s_ref[...], gws_ref[...], **_DOT)
              + jnp.dot(conv, gwc_ref[...], **_DOT) + gb_ref[...])
        a = 1.0 / (1.0 + jnp.exp(-gl))
        ns = a * s_ref[...] + (1.0 - a) * conv
        if final:
            dw_ref, db_ref, o_ref = rest
            o_ref[...] = jnp.dot(ns, dw_ref[...], **_DOT) + db_ref[...]
        else:
            i = pl.program_id(0)
            row = i * br + jax.lax.broadcasted_iota(jnp.int32, (br, 1), 0)
            o1_ref, o2_ref = rest
            o1_ref[...] = ns
            cb = jnp.where(row < n - npr, conv[:, d:], 0.0)
            o2_ref[...] = jax.lax.concatenate([conv[:, :d], cb], 1)

    full = lambda shp: pl.BlockSpec(shp, lambda i: tuple(0 for _ in shp))
    in_specs = [
        pl.BlockSpec((2, br, d2), lambda i: (0, i, 0)),
        pl.BlockSpec((br, d2), lambda i: (i, 0)),
        full((d2, d2)),
        full((1, d2)),
        full((d2, d2)),
        full((1, d2)),
        full((d2, d2)),
        full((d2, d2)),
        full((1, 1)),
    ]
    b2x = lambda b: jnp.concatenate([b, b]).reshape(1, d2)
    args = [p, s, _bd(w1), b2x(b1), _bd(w2), b2x(b2),
            _gate_mat(gw[:d], d), _gate_mat(gw[d:], d), gb.reshape(1, 1)]
    if final:
        d_out = dec_w.shape[1]
        in_specs += [full((d2, 2 * d_out)), full((1, 2 * d_out))]
        args += [_bd(dec_w), b2x(dec_b)]
        out_specs = pl.BlockSpec((br, 2 * d_out), lambda i: (i, 0))
        out_shape = jax.ShapeDtypeStruct((npr, 2 * d_out), jnp.float32)
    else:
        out_specs = [
            pl.BlockSpec((br, d2), lambda i: (i, 0)),
            pl.BlockSpec((br, d2), lambda i: (i, 0)),
        ]
        out_shape = [
            jax.ShapeDtypeStruct((npr, d2), jnp.float32),
            jax.ShapeDtypeStruct((npr, d2), jnp.float32),
        ]

    return pl.pallas_call(
        body,
        grid=(npr // br,),
        in_specs=in_specs,
        out_specs=out_specs,
        out_shape=out_shape,
    )(*args)


def kernel(x, edge_index, enc_W, enc_b, gin0_W1, gin0_b1, gin0_W2, gin0_b2,
           gin1_W1, gin1_b1, gin1_W2, gin1_b2, gate_W, gate_b, dec_W, dec_b):
    n = x.shape[0]
    d = enc_W.shape[1]
    e = edge_index.shape[1]
    nw = _NC * _NS
    np_ = ((n + 16 * 8 - 1) // (16 * 8)) * (16 * 8)
    npr = np_ // 2
    epw_r = e // nw
    grp = _CH * _K
    epw = ((epw_r + grp - 1) // grp) * grp
    ppw = epw - epw_r
    nch = epw // _CH
    src_pad = n + (jnp.arange(nw * ppw, dtype=jnp.int32) % (np_ - n))
    dst_pad = jnp.arange(nw * ppw, dtype=jnp.int32) % np_

    def remap(r):
        return jnp.where(r < npr, 2 * r, 2 * r - np_ + 1)

    pads = jnp.stack([src_pad, dst_pad]).reshape(2, nw, ppw)
    ei4 = remap(jnp.concatenate(
        [edge_index.reshape(2, nw, epw_r), pads],
        axis=2)).reshape(2, nw, nch, _CH)
    zeros = jnp.zeros((np_, d), jnp.float32)

    init_pair = _enc(x, enc_W, enc_b, np_)

    p0 = _seg_sum_partials(init_pair.reshape(np_, d), ei4, zeros,
                           np_).reshape(2, npr, 2 * d)
    self_pair, conv_pair = _mlp_gate(p0, init_pair, gin0_W1, gin0_b1,
                                     gin0_W2, gin0_b2, gate_W, gate_b, n)
    p1 = _seg_sum_partials(conv_pair.reshape(np_, d), ei4, zeros,
                           np_).reshape(2, npr, 2 * d)
    out_pair = _mlp_gate(p1, self_pair, gin1_W1, gin1_b1, gin1_W2, gin1_b2,
                         gate_W, gate_b, n, dec_W, dec_b)
    d_out = dec_W.shape[1]
    return jnp.concatenate([out_pair[:, :d_out], out_pair[:, d_out:]],
                           axis=0)[:n]


# speedup vs baseline: 4.6033x; 1.0224x over previous
import functools

import jax
import jax.numpy as jnp
from jax import lax
from jax.experimental import pallas as pl
from jax.experimental.pallas import tpu as pltpu
from jax.experimental.pallas import tpu_sc as plsc

_NC = 2
_NS = 16
_CH = 128
_K = 8


def _seg_sum_partials(h, ei4, zeros, np_):
    n, d = h.shape
    nch = ei4.shape[2]
    rpt = np_ // _NS

    mesh = plsc.VectorSubcoreMesh(core_axis_name="c", subcore_axis_name="s")

    @functools.partial(
        pl.kernel,
        mesh=mesh,
        compiler_params=pltpu.CompilerParams(use_tc_tiling_on_sc=False),
        out_type=jax.ShapeDtypeStruct((_NC * np_, d), jnp.float32),
        scratch_types=[
            pltpu.VMEM((nch, _CH), jnp.int32),
            pltpu.VMEM((nch, _CH), jnp.int32),
            pltpu.VMEM((_K, _CH, d), jnp.float32),
            pltpu.VMEM_SHARED((np_, d), jnp.float32),
            pltpu.SemaphoreType.DMA((_K,)),
            pltpu.SemaphoreType.DMA((_K,)),
        ],
    )
    def body(h_hbm, ei_hbm, z_hbm, out_hbm, sidx, didx, rows, acc,
             gsem, ssem):
        cid = lax.axis_index("c")
        sid = lax.axis_index("s")
        wid = sid * _NC + cid

        @pl.when(cid == 0)
        def _():
            pltpu.sync_copy(h_hbm.at[pl.ds(sid * rpt, rpt)],
                            acc.at[pl.ds(sid * rpt, rpt)])

        @pl.when(cid != 0)
        def _():
            pltpu.sync_copy(z_hbm.at[pl.ds(sid * rpt, rpt)],
                            acc.at[pl.ds(sid * rpt, rpt)])
        pltpu.sync_copy(ei_hbm.at[0, wid], sidx)
        pltpu.sync_copy(ei_hbm.at[1, wid], didx)

        def fire_gather(j, b):
            return pltpu.async_copy(h_hbm.at[sidx.at[j]], rows.at[b],
                                    gsem.at[b])

        def wait_gather(j, b):
            pltpu.make_async_copy(h_hbm.at[sidx.at[j]], rows.at[b],
                                  gsem.at[b]).wait()

        def fire_scatter(j, b):
            return pltpu.async_copy(rows.at[b], acc.at[didx.at[j]],
                                    ssem.at[b], add=True)

        def wait_scatter(j, b):
            pltpu.make_async_copy(rows.at[b], acc.at[didx.at[j]],
                                  ssem.at[b]).wait()

        for b in range(_K - 1):
            fire_gather(b, b)
        plsc.subcore_barrier()

        def group(m, carry):
            for b in range(_K):
                j = m * _K + b
                jj = j + _K - 1
                prev = (b - 1) % _K
                wait_gather(j, b)
                fire_scatter(j, b)

                @pl.when(jnp.logical_and(jj >= _K, jj < nch))
                def _():
                    wait_scatter(j - 1, prev)

                @pl.when(jnp.logical_and(jj >= _K - 1, jj < nch))
                def _():
                    fire_gather(jj, prev)
            return carry

        lax.fori_loop(0, nch // _K, group, 0)
        for i in range(_K):
            wait_scatter(nch - _K + i, i)
        plsc.subcore_barrier()
        pltpu.sync_copy(acc.at[pl.ds(sid * rpt, rpt)],
                        out_hbm.at[pl.ds(cid * np_ + sid * rpt, rpt)])

    return body(h, ei4, zeros)


_DOT = dict(preferred_element_type=jnp.float32)


def _enc(x, w, b, np_):
    n, d_in = x.shape
    d_h = w.shape[1]
    npr = np_ // 2
    br = npr // 8
    nb = 8

    def body(xt_ref, xb_ref, w_ref, b_ref, o_ref):
        i = pl.program_id(0)
        row = i * br + jax.lax.broadcasted_iota(jnp.int32, (br, 1), 0)
        ot = jnp.maximum(
            jnp.dot(xt_ref[...], w_ref[...], **_DOT) + b_ref[...], 0.0)
        ob = jnp.maximum(
            jnp.dot(xb_ref[...], w_ref[...], **_DOT) + b_ref[...], 0.0)
        ob = jnp.where(row < n - npr, ob, 0.0)
        o_ref[...] = jax.lax.concatenate([ot, ob], 1)

    return pl.pallas_call(
        body,
        grid=(nb,),
        in_specs=[
            pl.BlockSpec((br, d_in), lambda i: (i, 0)),
            pl.BlockSpec((br, d_in), lambda i: (i + nb, 0)),
            pl.BlockSpec((d_in, d_h), lambda i: (0, 0)),
            pl.BlockSpec((1, d_h), lambda i: (0, 0)),
        ],
        out_specs=pl.BlockSpec((br, 2 * d_h), lambda i: (i, 0)),
        out_shape=jax.ShapeDtypeStruct((npr, 2 * d_h), jnp.float32),
    )(x, x, w, b.reshape(1, d_h))


def _bd(w):
    z = jnp.zeros_like(w)
    return jnp.concatenate([jnp.concatenate([w, z], axis=1),
                            jnp.concatenate([z, w], axis=1)], axis=0)


def _gate_mat(gv, d):
    g = jnp.broadcast_to(gv, (d, d))
    z = jnp.zeros((d, d), jnp.float32)
    return jnp.concatenate([jnp.concatenate([g, z], axis=1),
                            jnp.concatenate([z, g], axis=1)], axis=0)


def _mlp_gate(p, s, w1, b1, w2, b2, gw, gb, n, dec_w=None, dec_b=None):
    npr, d2 = s.shape
    d = d2 // 2
    final = dec_w is not None
    br = npr // 8

    def body(p_ref, s_ref, w1_ref, b1_ref, w2_ref, b2_ref, gws_ref,
             gwc_ref, gb_ref, *rest):
        z = p_ref[0] + p_ref[1]
        t = jnp.maximum(jnp.dot(z, w1_ref[...], **_DOT) + b1_ref[...], 0.0)
        conv = jnp.dot(t, w2_ref[...], **_DOT) + b2_ref[...]
        gl = (jnp.dot(s_ref[...], gws_ref[...], **_DOT)
              + jnp.dot(conv, gwc_ref[...], **_DOT) + gb_ref[...])
        a = 1.0 / (1.0 + jnp.exp(-gl))
        ns = a * s_ref[...] + (1.0 - a) * conv
        if final:
            dw_ref, db_ref, o_ref = rest
            o_ref[0] = jnp.dot(ns[:, :d], dw_ref[...], **_DOT) + db_ref[...]
            o_ref[1] = jnp.dot(ns[:, d:], dw_ref[...], **_DOT) + db_ref[...]
        else:
            i = pl.program_id(0)
            row = i * br + jax.lax.broadcasted_iota(jnp.int32, (br, 1), 0)
            o1_ref, o2_ref = rest
            o1_ref[...] = ns
            cb = jnp.where(row < n - npr, conv[:, d:], 0.0)
            o2_ref[...] = jax.lax.concatenate([conv[:, :d], cb], 1)

    full = lambda shp: pl.BlockSpec(shp, lambda i: tuple(0 for _ in shp))
    in_specs = [
        pl.BlockSpec((2, br, d2), lambda i: (0, i, 0)),
        pl.BlockSpec((br, d2), lambda i: (i, 0)),
        full((d2, d2)),
        full((1, d2)),
        full((d2, d2)),
        full((1, d2)),
        full((d2, d2)),
        full((d2, d2)),
        full((1, 1)),
    ]
    b2x = lambda b: jnp.concatenate([b, b]).reshape(1, d2)
    args = [p, s, _bd(w1), b2x(b1), _bd(w2), b2x(b2),
            _gate_mat(gw[:d], d), _gate_mat(gw[d:], d), gb.reshape(1, 1)]
    if final:
        d_out = dec_w.shape[1]
        in_specs += [full((d, d_out)), full((1, d_out))]
        args += [dec_w, dec_b.reshape(1, d_out)]
        out_specs = pl.BlockSpec((2, br, d_out), lambda i: (0, i, 0))
        out_shape = jax.ShapeDtypeStruct((2, npr, d_out), jnp.float32)
    else:
        out_specs = [
            pl.BlockSpec((br, d2), lambda i: (i, 0)),
            pl.BlockSpec((br, d2), lambda i: (i, 0)),
        ]
        out_shape = [
            jax.ShapeDtypeStruct((npr, d2), jnp.float32),
            jax.ShapeDtypeStruct((npr, d2), jnp.float32),
        ]

    return pl.pallas_call(
        body,
        grid=(npr // br,),
        in_specs=in_specs,
        out_specs=out_specs,
        out_shape=out_shape,
    )(*args)


def kernel(x, edge_index, enc_W, enc_b, gin0_W1, gin0_b1, gin0_W2, gin0_b2,
           gin1_W1, gin1_b1, gin1_W2, gin1_b2, gate_W, gate_b, dec_W, dec_b):
    n = x.shape[0]
    d = enc_W.shape[1]
    e = edge_index.shape[1]
    nw = _NC * _NS
    np_ = ((n + 16 * 8 - 1) // (16 * 8)) * (16 * 8)
    npr = np_ // 2
    epw_r = e // nw
    grp = _CH * _K
    epw = ((epw_r + grp - 1) // grp) * grp
    ppw = epw - epw_r
    nch = epw // _CH
    src_pad = n + (jnp.arange(nw * ppw, dtype=jnp.int32) % (np_ - n))
    dst_pad = jnp.arange(nw * ppw, dtype=jnp.int32) % np_

    def remap(r):
        return jnp.where(r < npr, 2 * r, 2 * r - np_ + 1)

    pads = jnp.stack([src_pad, dst_pad]).reshape(2, nw, ppw)
    ei4 = remap(jnp.concatenate(
        [edge_index.reshape(2, nw, epw_r), pads],
        axis=2)).reshape(2, nw, nch, _CH)
    zeros = jnp.zeros((np_, d), jnp.float32)

    init_pair = _enc(x, enc_W, enc_b, np_)

    p0 = _seg_sum_partials(init_pair.reshape(np_, d), ei4, zeros,
                           np_).reshape(2, npr, 2 * d)
    self_pair, conv_pair = _mlp_gate(p0, init_pair, gin0_W1, gin0_b1,
                                     gin0_W2, gin0_b2, gate_W, gate_b, n)
    p1 = _seg_sum_partials(conv_pair.reshape(np_, d), ei4, zeros,
                           np_).reshape(2, npr, 2 * d)
    out3 = _mlp_gate(p1, self_pair, gin1_W1, gin1_b1, gin1_W2, gin1_b2,
                     gate_W, gate_b, n, dec_W, dec_b)
    return out3.reshape(2 * npr, dec_W.shape[1])[:n]


# speedup vs baseline: 4.8987x; 1.0642x over previous
import functools

import jax
import jax.numpy as jnp
from jax import lax
from jax.experimental import pallas as pl
from jax.experimental.pallas import tpu as pltpu
from jax.experimental.pallas import tpu_sc as plsc

_NC = 2
_NS = 16
_CH = 128
_K = 8


def _seg_sum_partials(h, ei4, zeros, np_):
    n, d = h.shape
    nch = ei4.shape[2]
    rpt = np_ // _NS

    mesh = plsc.VectorSubcoreMesh(core_axis_name="c", subcore_axis_name="s")

    @functools.partial(
        pl.kernel,
        mesh=mesh,
        compiler_params=pltpu.CompilerParams(use_tc_tiling_on_sc=False),
        out_type=jax.ShapeDtypeStruct((_NC * np_, d), jnp.float32),
        scratch_types=[
            pltpu.VMEM((nch, _CH), jnp.int32),
            pltpu.VMEM((nch, _CH), jnp.int32),
            pltpu.VMEM((_K, _CH, d), jnp.float32),
            pltpu.VMEM_SHARED((np_, d), jnp.float32),
            pltpu.SemaphoreType.DMA((_K,)),
            pltpu.SemaphoreType.DMA((_K,)),
        ],
    )
    def body(h_hbm, ei_hbm, z_hbm, out_hbm, sidx, didx, rows, acc,
             gsem, ssem):
        cid = lax.axis_index("c")
        sid = lax.axis_index("s")
        wid = sid * _NC + cid

        pltpu.sync_copy(ei_hbm.at[0, wid], sidx)
        pltpu.sync_copy(ei_hbm.at[1, wid], didx)

        def fire_gather(j, b):
            return pltpu.async_copy(h_hbm.at[sidx.at[j]], rows.at[b],
                                    gsem.at[b])

        def wait_gather(j, b):
            pltpu.make_async_copy(h_hbm.at[sidx.at[j]], rows.at[b],
                                  gsem.at[b]).wait()

        def fire_scatter(j, b):
            return pltpu.async_copy(rows.at[b], acc.at[didx.at[j]],
                                    ssem.at[b], add=True)

        def wait_scatter(j, b):
            pltpu.make_async_copy(rows.at[b], acc.at[didx.at[j]],
                                  ssem.at[b]).wait()

        for b in range(_K - 1):
            fire_gather(b, b)

        pltpu.sync_copy(z_hbm.at[pl.ds(sid * rpt, rpt)],
                        acc.at[pl.ds(sid * rpt, rpt)])
        plsc.subcore_barrier()

        def group(m, carry):
            for b in range(_K):
                j = m * _K + b
                jj = j + _K - 1
                prev = (b - 1) % _K
                wait_gather(j, b)
                fire_scatter(j, b)

                @pl.when(jnp.logical_and(jj >= _K, jj < nch))
                def _():
                    wait_scatter(j - 1, prev)

                @pl.when(jnp.logical_and(jj >= _K - 1, jj < nch))
                def _():
                    fire_gather(jj, prev)
            return carry

        lax.fori_loop(0, nch // _K, group, 0)
        for i in range(_K):
            wait_scatter(nch - _K + i, i)
        plsc.subcore_barrier()
        pltpu.sync_copy(acc.at[pl.ds(sid * rpt, rpt)],
                        out_hbm.at[pl.ds(cid * np_ + sid * rpt, rpt)])

    return body(h, ei4, zeros)


_DOT = dict(preferred_element_type=jnp.float32)


def _enc(x, w, b, np_):
    n, d_in = x.shape
    d_h = w.shape[1]
    npr = np_ // 2
    br = npr // 4
    nb = 4

    def body(xt_ref, xb_ref, w_ref, b_ref, o_ref):
        i = pl.program_id(0)
        row = i * br + jax.lax.broadcasted_iota(jnp.int32, (br, 1), 0)
        ot = jnp.maximum(
            jnp.dot(xt_ref[...], w_ref[...], **_DOT) + b_ref[...], 0.0)
        ob = jnp.maximum(
            jnp.dot(xb_ref[...], w_ref[...], **_DOT) + b_ref[...], 0.0)
        ob = jnp.where(row < n - npr, ob, 0.0)
        o_ref[...] = jax.lax.concatenate([ot, ob], 1)

    return pl.pallas_call(
        body,
        grid=(nb,),
        in_specs=[
            pl.BlockSpec((br, d_in), lambda i: (i, 0)),
            pl.BlockSpec((br, d_in), lambda i: (i + nb, 0)),
            pl.BlockSpec((d_in, d_h), lambda i: (0, 0)),
            pl.BlockSpec((1, d_h), lambda i: (0, 0)),
        ],
        out_specs=pl.BlockSpec((br, 2 * d_h), lambda i: (i, 0)),
        out_shape=jax.ShapeDtypeStruct((npr, 2 * d_h), jnp.float32),
    )(x, x, w, b.reshape(1, d_h))


def _bd(w):
    z = jnp.zeros_like(w)
    return jnp.concatenate([jnp.concatenate([w, z], axis=1),
                            jnp.concatenate([z, w], axis=1)], axis=0)


def _mlp_gate(p, h, s, w1, b1, w2, b2, gw, gb, n, dec_w=None, dec_b=None):
    npr, d2 = s.shape
    d = d2 // 2
    final = dec_w is not None
    br = npr // 4

    def body(p_ref, h_ref, s_ref, w1_ref, b1_ref, w2_ref, b2_ref, gw_ref,
             gb_ref, *rest):
        z = (p_ref[0] + p_ref[1]) + h_ref[...]
        t = jnp.maximum(jnp.dot(z, w1_ref[...], **_DOT) + b1_ref[...], 0.0)
        conv = jnp.dot(t, w2_ref[...], **_DOT) + b2_ref[...]
        s_v = s_ref[...]
        a_halves = []
        for lo, hi in ((0, d), (d, d2)):
            cat = jax.lax.concatenate([s_v[:, lo:hi], conv[:, lo:hi]], 1)
            gl = jnp.dot(cat, gw_ref[...], **_DOT) + gb_ref[...]
            a_halves.append(jnp.broadcast_to(jax.nn.sigmoid(gl), (br, d)))
        a = jax.lax.concatenate(a_halves, 1)
        ns = a * s_v + (1.0 - a) * conv
        if final:
            dw_ref, db_ref, o_ref = rest
            o_ref[0] = jnp.dot(ns[:, :d], dw_ref[...], **_DOT) + db_ref[...]
            o_ref[1] = jnp.dot(ns[:, d:], dw_ref[...], **_DOT) + db_ref[...]
        else:
            i = pl.program_id(0)
            row = i * br + jax.lax.broadcasted_iota(jnp.int32, (br, 1), 0)
            o1_ref, o2_ref = rest
            o1_ref[...] = ns
            cb = jnp.where(row < n - npr, conv[:, d:], 0.0)
            o2_ref[...] = jax.lax.concatenate([conv[:, :d], cb], 1)

    full = lambda shp: pl.BlockSpec(shp, lambda i: tuple(0 for _ in shp))
    in_specs = [
        pl.BlockSpec((2, br, d2), lambda i: (0, i, 0)),
        pl.BlockSpec((br, d2), lambda i: (i, 0)),
        pl.BlockSpec((br, d2), lambda i: (i, 0)),
        full((d2, d2)),
        full((1, d2)),
        full((d2, d2)),
        full((1, d2)),
        full((d2, 1)),
        full((1, 1)),
    ]
    b2x = lambda b: jnp.concatenate([b, b]).reshape(1, d2)
    args = [p, h, s, _bd(w1), b2x(b1), _bd(w2), b2x(b2),
            gw, gb.reshape(1, 1)]
    if final:
        d_out = dec_w.shape[1]
        in_specs += [full((d, d_out)), full((1, d_out))]
        args += [dec_w, dec_b.reshape(1, d_out)]
        out_specs = pl.BlockSpec((2, br, d_out), lambda i: (0, i, 0))
        out_shape = jax.ShapeDtypeStruct((2, npr, d_out), jnp.float32)
    else:
        out_specs = [
            pl.BlockSpec((br, d2), lambda i: (i, 0)),
            pl.BlockSpec((br, d2), lambda i: (i, 0)),
        ]
        out_shape = [
            jax.ShapeDtypeStruct((npr, d2), jnp.float32),
            jax.ShapeDtypeStruct((npr, d2), jnp.float32),
        ]

    return pl.pallas_call(
        body,
        grid=(npr // br,),
        in_specs=in_specs,
        out_specs=out_specs,
        out_shape=out_shape,
    )(*args)


def kernel(x, edge_index, enc_W, enc_b, gin0_W1, gin0_b1, gin0_W2, gin0_b2,
           gin1_W1, gin1_b1, gin1_W2, gin1_b2, gate_W, gate_b, dec_W, dec_b):
    n = x.shape[0]
    d = enc_W.shape[1]
    e = edge_index.shape[1]
    nw = _NC * _NS
    np_ = ((n + 16 * 8 - 1) // (16 * 8)) * (16 * 8)
    npr = np_ // 2
    epw_r = e // nw
    grp = _CH * _K
    epw = ((epw_r + grp - 1) // grp) * grp
    ppw = epw - epw_r
    nch = epw // _CH
    src_pad = n + (jnp.arange(nw * ppw, dtype=jnp.int32) % (np_ - n))
    dst_pad = jnp.arange(nw * ppw, dtype=jnp.int32) % np_

    def remap(r):
        return jnp.where(r < npr, 2 * r, 2 * r - np_ + 1)

    pads = jnp.stack([src_pad, dst_pad]).reshape(2, nw, ppw)
    ei4 = remap(jnp.concatenate(
        [edge_index.reshape(2, nw, epw_r), pads],
        axis=2)).reshape(2, nw, nch, _CH)
    zeros = jnp.zeros((np_, d), jnp.float32)

    init_pair = _enc(x, enc_W, enc_b, np_)

    p0 = _seg_sum_partials(init_pair.reshape(np_, d), ei4, zeros,
                           np_).reshape(2, npr, 2 * d)
    self_pair, conv_pair = _mlp_gate(p0, init_pair, init_pair, gin0_W1,
                                     gin0_b1, gin0_W2, gin0_b2, gate_W,
                                     gate_b, n)
    p1 = _seg_sum_partials(conv_pair.reshape(np_, d), ei4, zeros,
                           np_).reshape(2, npr, 2 * d)
    out3 = _mlp_gate(p1, conv_pair, self_pair, gin1_W1, gin1_b1, gin1_W2,
                     gin1_b2, gate_W, gate_b, n, dec_W, dec_b)
    return out3.reshape(2 * npr, dec_W.shape[1])[:n]
